# baseline (speedup 1.0000x reference)
import dataclasses

import jax
import jax.numpy as jnp
from jax import lax
from jax.experimental import pallas as pl
from jax.experimental.pallas import tpu as pltpu
from jax.experimental.pallas import tpu_sc as plsc

_NW = 32


def kernel(onset_idx, rhyme_idx, tone_idx, onset_table, rhyme_table, tone_table):
    B, L = onset_idx.shape
    V, D = onset_table.shape
    n = B * L * 3
    bw = B // _NW
    win = (bw // 8) * 24

    mesh = plsc.VectorSubcoreMesh(core_axis_name="c", subcore_axis_name="s")
    cp = pltpu.CompilerParams()
    if "needs_layout_passes" in pltpu.CompilerParams.__dataclass_fields__:
        cp = dataclasses.replace(cp, needs_layout_passes=False)

    @pl.kernel(
        out_type=jax.ShapeDtypeStruct((n, D), jnp.float32),
        mesh=mesh,
        compiler_params=cp,
        scratch_types=[
            pltpu.VMEM_SHARED((3 * V, D), jnp.float32),
            pltpu.VMEM((bw, L), jnp.int32),
            pltpu.VMEM((L * win,), jnp.int32),
            pltpu.VMEM((win // 4, D), jnp.float32),
            pltpu.VMEM((win // 4, D), jnp.float32),
            pltpu.VMEM((win // 4, D), jnp.float32),
            pltpu.VMEM((win // 4, D), jnp.float32),
            pltpu.SemaphoreType.DMA,
            pltpu.SemaphoreType.DMA,
            pltpu.SemaphoreType.DMA,
            pltpu.SemaphoreType.DMA,
            pltpu.SemaphoreType.DMA,
            pltpu.SemaphoreType.DMA,
            pltpu.SemaphoreType.DMA,
            pltpu.SemaphoreType.DMA,
        ],
    )
    def gather_kernel(
        i1_hbm, i2_hbm, i3_hbm, t1_hbm, t2_hbm, t3_hbm, o_hbm,
        tab_spmem, raw_v, perm_v, buf0, buf1, buf2, buf3,
        gsem0, gsem1, gsem2, gsem3, wsem0, wsem1, wsem2, wsem3,
    ):
        cid = lax.axis_index("c")
        sid = lax.axis_index("s")
        wid = cid * 16 + sid
        b0 = wid * bw

        for t, t_hbm in enumerate((t1_hbm, t2_hbm, t3_hbm)):

            @pl.when(sid == t)
            def _(t=t, t_hbm=t_hbm):
                pltpu.make_async_copy(
                    t_hbm, tab_spmem.at[pl.ds(t * V, V)], gsem0
                ).start()

        lane = lax.iota(jnp.int32, 16)
        dst_pat = (lane // 8) * 24 + (lane % 8)
        for t, i_hbm in enumerate((i1_hbm, i2_hbm, i3_hbm)):
            pltpu.sync_copy(i_hbm.at[pl.ds(b0, bw)], raw_v)

            @pl.loop(0, L)
            def _(l):
                for j in range(bw // 16):
                    rows = j * 16 + lane
                    cols = jnp.full((16,), l, jnp.int32)
                    v = plsc.load_gather(raw_v, [rows, cols]) + t * V
                    dst = l * win + j * 48 + t * 8 + dst_pat
                    plsc.store_scatter(perm_v, [dst], v)

        @pl.when(sid < 3)
        def _():
            pltpu.make_async_copy(
                t1_hbm, tab_spmem.at[pl.ds(0, V)], gsem0
            ).wait()

        plsc.subcore_barrier()

        q = win // 4
        bufs = (buf0, buf1, buf2, buf3)
        gsems = (gsem0, gsem1, gsem2, gsem3)
        wsems = (wsem0, wsem1, wsem2, wsem3)
        nsteps = 4 * L

        def gstart(s, buf, gsem):
            pltpu.make_async_copy(
                tab_spmem.at[perm_v.at[pl.ds(s * q, q)]], buf, gsem
            ).start()

        def gwait(buf, gsem):
            pltpu.make_async_copy(
                tab_spmem.at[perm_v.at[pl.ds(0, q)]], buf, gsem
            ).wait()

        def wstart(s, buf, wsem):
            p0 = (s // 4) * (B // 8) * 24 + wid * win + (s % 4) * q
            pltpu.make_async_copy(buf, o_hbm.at[pl.ds(p0, q)], wsem).start()

        def wwait(buf, wsem):
            pltpu.make_async_copy(buf, o_hbm.at[pl.ds(0, q)], wsem).wait()

        for k in range(4):
            gstart(k, bufs[k], gsems[k])

        @pl.loop(0, nsteps, step=4)
        def _(s):
            for k in range(4):
                gwait(bufs[k], gsems[k])
                wstart(s + k, bufs[k], wsems[k])
            for k in range(4):

                @pl.when(s + 4 + k < nsteps)
                def _(k=k):
                    wwait(bufs[k], wsems[k])
                    gstart(s + 4 + k, bufs[k], gsems[k])

        for k in range(4):
            wwait(bufs[k], wsems[k])

    rows = gather_kernel(
        onset_idx.astype(jnp.int32),
        rhyme_idx.astype(jnp.int32),
        tone_idx.astype(jnp.int32),
        onset_table,
        rhyme_table,
        tone_table,
    )
    out = (
        rows.reshape(L, B // 8, 3, 8, D)
        .transpose(1, 3, 0, 2, 4)
        .reshape(B, L, 3 * D)
    )
    return out
